# baseline (speedup 1.0000x reference)
import functools
import math

import jax
import jax.numpy as jnp
from jax.experimental import pallas as pl
from jax.experimental.pallas import tpu as pltpu

_INV_SQRT2 = 1.0 / math.sqrt(2.0)
_NBI = 2
_NBO = 2
_K = 2


def _moe_block(bt, x_hbm, wr_ref, br_ref, w1_ref, b1_ref, w2_ref, b2_ref,
               o_hbm, xb, ob, in_sem, out_sem):
    n = pl.program_id(0)
    nsteps = pl.num_programs(0)
    sub = bt // _K

    def copy_in(step, slot, k):
        rows = pl.ds(step * bt + k * sub, sub)
        return pltpu.make_async_copy(
            x_hbm.at[rows, :], xb.at[slot, pl.ds(k * sub, sub), :],
            in_sem.at[slot, k],
        )

    def copy_out(step, slot, k):
        rows = pl.ds(step * bt + k * sub, sub)
        return pltpu.make_async_copy(
            ob.at[slot, pl.ds(k * sub, sub), :], o_hbm.at[rows, :],
            out_sem.at[slot, k],
        )

    def start_in(step, slot):
        for k in range(_K):
            copy_in(step, slot, k).start()

    def wait_in(step, slot):
        for k in range(_K):
            copy_in(step, slot, k).wait()

    def start_out(step, slot):
        for k in range(_K):
            copy_out(step, slot, k).start()

    def wait_out(step, slot):
        for k in range(_K):
            copy_out(step, slot, k).wait()

    @pl.when(n == 0)
    def _prologue():
        for s in range(_NBI):
            start_in(s, s)

    islot = jax.lax.rem(n, _NBI)
    oslot = jax.lax.rem(n, _NBO)

    @pl.when(n >= _NBO)
    def _drain_out():
        wait_out(n - _NBO, oslot)

    @pl.when(n > 0)
    def _wait_block():
        wait_in(n, islot)

    for k in range(_K):
        rows = pl.ds(k * sub, sub)

        @pl.when(n == 0)
        def _wait_chunk():
            copy_in(n, islot, k).wait()
        x = xb[islot, rows, :]
        logits = (
            jnp.dot(x, wr_ref[...], preferred_element_type=jnp.float32)
            + br_ref[...]
        )
        m = jnp.max(logits, axis=-1, keepdims=True)
        e = jnp.exp(logits - m)
        router = e / jnp.sum(e, axis=-1, keepdims=True)
        h = (
            jnp.dot(router, w1_ref[...], preferred_element_type=jnp.float32)
            + b1_ref[...]
        )
        g = h * 0.5 * (1.0 + jax.lax.erf(h * _INV_SQRT2))
        ob[oslot, rows, :] = (
            jnp.dot(g, w2_ref[...], preferred_element_type=jnp.float32)
            + b2_ref[...]
        )
        copy_out(n, oslot, k).start()

    @pl.when(n + _NBI < nsteps)
    def _prefetch():
        start_in(n + _NBI, islot)

    @pl.when(n == nsteps - 1)
    def _epilogue():
        for j in range(_NBO):
            step = n - (_NBO - 1) + j
            wait_out(step, jax.lax.rem(step, _NBO))


@functools.partial(jax.jit, static_argnames=("block_tokens", "interpret"))
def _moe_fused(x2d, Wr, br, W1l, b1l, W2l, b2l, block_tokens, interpret=False):
    n_tok, dim = x2d.shape
    e = Wr.shape[1]
    inner = W1l.shape[1]
    nsteps = n_tok // block_tokens
    full = lambda shape: pl.BlockSpec(shape, lambda i: (0,) * len(shape))
    hbm_spec = pl.BlockSpec(memory_space=pltpu.MemorySpace.HBM)
    return pl.pallas_call(
        functools.partial(_moe_block, block_tokens),
        grid=(nsteps,),
        in_specs=[
            hbm_spec,
            full((dim, e)),
            full((1, e)),
            full((e, inner)),
            full((1, inner)),
            full((inner, dim)),
            full((1, dim)),
        ],
        out_specs=hbm_spec,
        out_shape=jax.ShapeDtypeStruct((n_tok, dim), x2d.dtype),
        scratch_shapes=[
            pltpu.VMEM((_NBI, block_tokens, dim), jnp.float32),
            pltpu.VMEM((_NBO, block_tokens, dim), jnp.float32),
            pltpu.SemaphoreType.DMA((_NBI, _K)),
            pltpu.SemaphoreType.DMA((_NBO, _K)),
        ],
        compiler_params=pltpu.CompilerParams(
            dimension_semantics=("arbitrary",),
        ),
        interpret=interpret,
    )(x2d, Wr, br, W1l, b1l, W2l, b2l)


def kernel(x, Wr, br, W1, b1, W2, b2):
    B, S, DIM = x.shape
    x2d = x.reshape(B * S, DIM)
    out = _moe_fused(
        x2d,
        Wr,
        br.reshape(1, -1),
        W1[-1],
        b1[-1].reshape(1, -1),
        W2[-1],
        b2[-1].reshape(1, -1),
        block_tokens=4096,
    )
    return out.reshape(B, S, DIM)


# speedup vs baseline: 1.0555x; 1.0555x over previous
import functools
import math

import jax
import jax.numpy as jnp
from jax.experimental import pallas as pl
from jax.experimental.pallas import tpu as pltpu

_INV_SQRT2 = 1.0 / math.sqrt(2.0)
_NBI = 2
_NBO = 2
_K = 2


def _moe_block(bt, x_hbm, wr_ref, br_ref, w1_ref, b1_ref, w2_ref, b2_ref,
               o_hbm, xb, ob, in_sem, out_sem):
    n = pl.program_id(0)
    nsteps = pl.num_programs(0)
    sub = bt // _K

    def copy_in(step, slot, k):
        rows = pl.ds(step * bt + k * sub, sub)
        return pltpu.make_async_copy(
            x_hbm.at[rows, :], xb.at[slot, pl.ds(k * sub, sub), :],
            in_sem.at[slot, k],
        )

    def copy_out(step, slot, k):
        rows = pl.ds(step * bt + k * sub, sub)
        return pltpu.make_async_copy(
            ob.at[slot, pl.ds(k * sub, sub), :], o_hbm.at[rows, :],
            out_sem.at[slot, k],
        )

    def start_in(step, slot):
        for k in range(_K):
            copy_in(step, slot, k).start()

    def wait_in(step, slot):
        for k in range(_K):
            copy_in(step, slot, k).wait()

    def start_out(step, slot):
        for k in range(_K):
            copy_out(step, slot, k).start()

    def wait_out(step, slot):
        for k in range(_K):
            copy_out(step, slot, k).wait()

    @pl.when(n == 0)
    def _prologue():
        for s in range(_NBI):
            start_in(s, s)

    islot = jax.lax.rem(n, _NBI)
    oslot = jax.lax.rem(n, _NBO)

    @pl.when(n >= _NBO)
    def _drain_out():
        wait_out(n - _NBO, oslot)

    wait_in(n, islot)

    for k in range(_K):
        rows = pl.ds(k * sub, sub)
        x = xb[islot, rows, :]
        logits = (
            jnp.dot(x, wr_ref[...], preferred_element_type=jnp.float32)
            + br_ref[...]
        )
        m = jnp.max(logits, axis=-1, keepdims=True)
        e = jnp.exp(logits - m)
        router = e / jnp.sum(e, axis=-1, keepdims=True)
        h = (
            jnp.dot(router, w1_ref[...], preferred_element_type=jnp.float32)
            + b1_ref[...]
        )
        g = h * 0.5 * (1.0 + jax.lax.erf(h * _INV_SQRT2))
        ob[oslot, rows, :] = (
            jnp.dot(g, w2_ref[...], preferred_element_type=jnp.float32)
            + b2_ref[...]
        )
        copy_out(n, oslot, k).start()

        @pl.when(n + _NBI < nsteps)
        def _prefetch():
            copy_in(n + _NBI, islot, k).start()

    @pl.when(n == nsteps - 1)
    def _epilogue():
        for j in range(_NBO):
            step = n - (_NBO - 1) + j
            wait_out(step, jax.lax.rem(step, _NBO))


@functools.partial(jax.jit, static_argnames=("block_tokens", "interpret"))
def _moe_fused(x2d, Wr, br, W1l, b1l, W2l, b2l, block_tokens, interpret=False):
    n_tok, dim = x2d.shape
    e = Wr.shape[1]
    inner = W1l.shape[1]
    nsteps = n_tok // block_tokens
    full = lambda shape: pl.BlockSpec(shape, lambda i: (0,) * len(shape))
    hbm_spec = pl.BlockSpec(memory_space=pltpu.MemorySpace.HBM)
    return pl.pallas_call(
        functools.partial(_moe_block, block_tokens),
        grid=(nsteps,),
        in_specs=[
            hbm_spec,
            full((dim, e)),
            full((1, e)),
            full((e, inner)),
            full((1, inner)),
            full((inner, dim)),
            full((1, dim)),
        ],
        out_specs=hbm_spec,
        out_shape=jax.ShapeDtypeStruct((n_tok, dim), x2d.dtype),
        scratch_shapes=[
            pltpu.VMEM((_NBI, block_tokens, dim), jnp.float32),
            pltpu.VMEM((_NBO, block_tokens, dim), jnp.float32),
            pltpu.SemaphoreType.DMA((_NBI, _K)),
            pltpu.SemaphoreType.DMA((_NBO, _K)),
        ],
        compiler_params=pltpu.CompilerParams(
            dimension_semantics=("arbitrary",),
        ),
        interpret=interpret,
    )(x2d, Wr, br, W1l, b1l, W2l, b2l)


def kernel(x, Wr, br, W1, b1, W2, b2):
    B, S, DIM = x.shape
    x2d = x.reshape(B * S, DIM)
    out = _moe_fused(
        x2d,
        Wr,
        br.reshape(1, -1),
        W1[-1],
        b1[-1].reshape(1, -1),
        W2[-1],
        b2[-1].reshape(1, -1),
        block_tokens=4096,
    )
    return out.reshape(B, S, DIM)


# speedup vs baseline: 1.0921x; 1.0347x over previous
import functools
import math

import jax
import jax.numpy as jnp
from jax.experimental import pallas as pl
from jax.experimental.pallas import tpu as pltpu

_INV_SQRT2 = 1.0 / math.sqrt(2.0)
_NBI = 2
_NBO = 2
_K = 4


def _moe_block(bt, x_hbm, wr_ref, br_ref, w1_ref, b1_ref, w2_ref, b2_ref,
               o_hbm, xb, ob, in_sem, out_sem):
    n = pl.program_id(0)
    nsteps = pl.num_programs(0)
    sub = bt // _K

    def copy_in(step, slot, k):
        rows = pl.ds(step * bt + k * sub, sub)
        return pltpu.make_async_copy(
            x_hbm.at[rows, :], xb.at[slot, pl.ds(k * sub, sub), :],
            in_sem.at[slot, k],
        )

    def copy_out(step, slot, k):
        rows = pl.ds(step * bt + k * sub, sub)
        return pltpu.make_async_copy(
            ob.at[slot, pl.ds(k * sub, sub), :], o_hbm.at[rows, :],
            out_sem.at[slot, k],
        )

    def start_in(step, slot):
        for k in range(_K):
            copy_in(step, slot, k).start()

    def wait_in(step, slot):
        for k in range(_K):
            copy_in(step, slot, k).wait()

    def start_out(step, slot):
        for k in range(_K):
            copy_out(step, slot, k).start()

    def wait_out(step, slot):
        for k in range(_K):
            copy_out(step, slot, k).wait()

    @pl.when(n == 0)
    def _prologue():
        for s in range(_NBI):
            start_in(s, s)

    islot = jax.lax.rem(n, _NBI)
    oslot = jax.lax.rem(n, _NBO)

    @pl.when(n >= _NBO)
    def _drain_out():
        wait_out(n - _NBO, oslot)

    wait_in(n, islot)

    for k in range(_K):
        rows = pl.ds(k * sub, sub)
        x = xb[islot, rows, :]
        logits = (
            jnp.dot(x, wr_ref[...], preferred_element_type=jnp.float32)
            + br_ref[...]
        )
        m = jnp.max(logits, axis=-1, keepdims=True)
        e = jnp.exp(logits - m)
        router = e / jnp.sum(e, axis=-1, keepdims=True)
        h = (
            jnp.dot(router, w1_ref[...], preferred_element_type=jnp.float32)
            + b1_ref[...]
        )
        g = h * 0.5 * (1.0 + jax.lax.erf(h * _INV_SQRT2))
        ob[oslot, rows, :] = (
            jnp.dot(g, w2_ref[...], preferred_element_type=jnp.float32)
            + b2_ref[...]
        )
        copy_out(n, oslot, k).start()

        @pl.when(n + _NBI < nsteps)
        def _prefetch():
            copy_in(n + _NBI, islot, k).start()

    @pl.when(n == nsteps - 1)
    def _epilogue():
        for j in range(_NBO):
            step = n - (_NBO - 1) + j
            wait_out(step, jax.lax.rem(step, _NBO))


@functools.partial(jax.jit, static_argnames=("block_tokens", "interpret"))
def _moe_fused(x2d, Wr, br, W1l, b1l, W2l, b2l, block_tokens, interpret=False):
    n_tok, dim = x2d.shape
    e = Wr.shape[1]
    inner = W1l.shape[1]
    nsteps = n_tok // block_tokens
    full = lambda shape: pl.BlockSpec(shape, lambda i: (0,) * len(shape))
    hbm_spec = pl.BlockSpec(memory_space=pltpu.MemorySpace.HBM)
    return pl.pallas_call(
        functools.partial(_moe_block, block_tokens),
        grid=(nsteps,),
        in_specs=[
            hbm_spec,
            full((dim, e)),
            full((1, e)),
            full((e, inner)),
            full((1, inner)),
            full((inner, dim)),
            full((1, dim)),
        ],
        out_specs=hbm_spec,
        out_shape=jax.ShapeDtypeStruct((n_tok, dim), x2d.dtype),
        scratch_shapes=[
            pltpu.VMEM((_NBI, block_tokens, dim), jnp.float32),
            pltpu.VMEM((_NBO, block_tokens, dim), jnp.float32),
            pltpu.SemaphoreType.DMA((_NBI, _K)),
            pltpu.SemaphoreType.DMA((_NBO, _K)),
        ],
        compiler_params=pltpu.CompilerParams(
            dimension_semantics=("arbitrary",),
        ),
        interpret=interpret,
    )(x2d, Wr, br, W1l, b1l, W2l, b2l)


def kernel(x, Wr, br, W1, b1, W2, b2):
    B, S, DIM = x.shape
    x2d = x.reshape(B * S, DIM)
    out = _moe_fused(
        x2d,
        Wr,
        br.reshape(1, -1),
        W1[-1],
        b1[-1].reshape(1, -1),
        W2[-1],
        b2[-1].reshape(1, -1),
        block_tokens=4096,
    )
    return out.reshape(B, S, DIM)
